# baseline (speedup 1.0000x reference)
import functools

import jax
import jax.numpy as jnp
from jax import lax
from jax.experimental import pallas as pl
from jax.experimental.pallas import tpu as pltpu
from jax.experimental.pallas import tpu_sc as plsc

_D = 128


def _gelu(x):
    return 0.5 * x * (1.0 + lax.erf(x * (2.0 ** -0.5)))


@functools.lru_cache(maxsize=None)
def _make_sc_gather(Qn, B):
    info = plsc.get_sparse_core_info()
    NC, NS = info.num_cores, info.num_subcores
    NW = NC * NS
    assert B % (8 * NW) == 0
    b_per_w = B // NW
    mesh = plsc.VectorSubcoreMesh(core_axis_name="c", subcore_axis_name="s")

    @functools.partial(
        pl.kernel,
        mesh=mesh,
        out_type=jax.ShapeDtypeStruct((B, _D), jnp.float32),
        scratch_types=[
            pltpu.VMEM((b_per_w,), jnp.int32),
            pltpu.VMEM((b_per_w, _D), jnp.float32),
            pltpu.SemaphoreType.DMA,
        ],
    )
    def gather_k(tbl_hbm, idx_hbm, out_hbm, idx_v, rows_v, sem):
        wid = lax.axis_index("s") * NC + lax.axis_index("c")
        base = wid * b_per_w
        pltpu.sync_copy(idx_hbm.at[pl.ds(base, b_per_w)], idx_v)
        pltpu.async_copy(tbl_hbm.at[idx_v], rows_v, sem).wait()
        pltpu.sync_copy(rows_v, out_hbm.at[pl.ds(base, b_per_w)])

    return gather_k


def _tc_body(Un, Tn, Kn, g_ref, resp_ref, W1_ref, b1_ref,
             W2_ref, b2_ref, W3_ref, b3_ref,
             logits_ref, last_ref,
             lm_ref, lmmu_ref, pre_ref, rden_ref, cm_ref, icnt_ref):
    B = Un * Tn
    C = 4096
    TC = C // Un
    f32 = jnp.float32
    Hn = W2_ref.shape[0]

    scale = 1.0 / Kn
    W1cat = jnp.concatenate(
        [jnp.broadcast_to(W1_ref[0:1, :] * scale, (Kn, Hn)),
         jnp.broadcast_to(W1_ref[1:2, :] * scale, (Kn, Hn)),
         jnp.broadcast_to(W1_ref[2:3, :] * scale, (Kn, Hn))], axis=0)
    W3cat = jnp.concatenate(
        [jnp.broadcast_to(W3_ref[:, 0:1], (Hn, Kn)),
         jnp.broadcast_to(W3_ref[:, 1:2], (Hn, Kn))], axis=1)
    b3cat = jnp.concatenate(
        [jnp.broadcast_to(b3_ref[0:1, 0:1], (1, Kn)),
         jnp.broadcast_to(b3_ref[0:1, 1:2], (1, Kn))], axis=1)

    ones_k = jnp.ones((Kn, Kn), f32)
    ones_uk = jnp.ones((Un, Kn), f32)
    rrow = lax.broadcasted_iota(jnp.int32, (C, TC), 0) // Un
    rcol = lax.broadcasted_iota(jnp.int32, (C, TC), 1)
    sel = (rrow == rcol).astype(f32)
    urow = lax.broadcasted_iota(jnp.int32, (C, Un), 0) % Un
    ucol = lax.broadcasted_iota(jnp.int32, (C, Un), 1)
    eye_u = (urow == ucol).astype(f32)

    b1 = b1_ref[...]
    b2 = b2_ref[...]

    for c in range(B // C):
        r0 = c * C
        rows = pl.ds(r0, C)
        mch = g_ref[rows, 0:Kn]
        rsel = jnp.dot(sel, resp_ref[pl.ds(c * TC, TC), :],
                       preferred_element_type=f32)
        r16 = jnp.dot(rsel * eye_u, ones_uk, preferred_element_type=f32)
        xfull = jnp.concatenate([g_ref[rows, Kn:3 * Kn], r16], axis=1)
        h = _gelu(jnp.dot(xfull, W1cat, preferred_element_type=f32) + b1)
        h = _gelu(jnp.dot(h, W2_ref[...], preferred_element_type=f32) + b2)
        o = jnp.dot(h, W3cat, preferred_element_type=f32) + b3cat
        mu16 = _gelu(o[:, 0:Kn])
        lv16 = jnp.minimum(_gelu(o[:, Kn:2 * Kn]), 1e8)
        lm16 = jnp.exp(-lv16)
        lm_ref[rows, :] = lm16
        lmmu_ref[rows, :] = lm16 * mu16
        cnt = jnp.dot(mch, ones_k, preferred_element_type=f32)
        icnt_ref[rows, :] = 1.0 / jnp.maximum(cnt, 1e-8)

    zeros = jnp.zeros((Un, Kn), f32)

    def bwd(i, carry):
        al, be = carry
        t = (Tn - 1) - i
        rows = pl.ds(pl.multiple_of(t * Un, Un), Un)
        m = g_ref[rows, 0:Kn] != 0.0
        lm = lm_ref[rows, :]
        lmmu = lmmu_ref[rows, :]
        num = lmmu + al * be
        den = lm + al
        rf = 1.0 / (1.0 + den)
        pre_ref[rows, :] = num
        rden_ref[rows, :] = rf
        al_new = den * rf
        be_new = num / den
        return (jnp.where(m, al_new, al), jnp.where(m, be_new, be))

    lax.fori_loop(0, Tn, bwd, (zeros, zeros), unroll=8)

    def fwd(t, curr):
        rows = pl.ds(pl.multiple_of(t * Un, Un), Un)
        mf = g_ref[rows, 0:Kn]
        cand = (curr + pre_ref[rows, :]) * rden_ref[rows, :]
        curr = jnp.where(mf != 0.0, cand, curr)
        cm_ref[rows, :] = curr * mf
        return curr

    last_ref[...] = lax.fori_loop(0, Tn, fwd, zeros, unroll=8)

    for c in range(B // C):
        rows = pl.ds(c * C, C)
        ssum = jnp.dot(cm_ref[rows, :], ones_k, preferred_element_type=f32)
        abil = ssum * icnt_ref[rows, :]
        d16 = g_ref[rows, Kn:2 * Kn]
        s16 = g_ref[rows, 2 * Kn:3 * Kn]
        logits_ref[rows, :] = (s16 * (abil - d16))[:, 0:1]


def kernel(mask, q_id, kmap, resp, diff_mu, disc_mu, W1, b1, W2, b2, W3, b3):
    Un, Tn = q_id.shape
    Qn, Kn = kmap.shape
    Hn = W1.shape[1]
    B = Un * Tn
    f32 = jnp.float32

    ci = jnp.arange(_D, dtype=jnp.int32)[None, :]
    kpad = jnp.pad(kmap, ((0, 0), (0, _D - Kn))).astype(f32)
    tbl = jnp.where(ci < Kn, kpad,
                    jnp.where(ci < 2 * Kn, diff_mu[:, None],
                              jnp.where(ci < 3 * Kn, disc_mu[:, None], 0.0)))
    idx = q_id.astype(jnp.int32).T.reshape(B)
    g = _make_sc_gather(Qn, B)(tbl, idx)

    resp_t = resp.astype(f32).T

    logits_col, last = pl.pallas_call(
        functools.partial(_tc_body, Un, Tn, Kn),
        out_shape=(jax.ShapeDtypeStruct((B, 1), f32),
                   jax.ShapeDtypeStruct((Un, Kn), f32)),
        scratch_shapes=[pltpu.VMEM((B, Kn), f32)] * 6,
    )(g, resp_t, W1, b1.reshape(1, Hn), W2, b2.reshape(1, Hn),
      W3, b3.reshape(1, 2))

    trial_logits = logits_col.reshape(Tn, Un).T
    return (trial_logits, last)


# speedup vs baseline: 1.0210x; 1.0210x over previous
import functools

import jax
import jax.numpy as jnp
from jax import lax
from jax.experimental import pallas as pl
from jax.experimental.pallas import tpu as pltpu
from jax.experimental.pallas import tpu_sc as plsc

_D = 128


def _gelu(x):
    return 0.5 * x * (1.0 + lax.erf(x * (2.0 ** -0.5)))


@functools.lru_cache(maxsize=None)
def _make_sc_gather(Qn, B):
    info = plsc.get_sparse_core_info()
    NC, NS = info.num_cores, info.num_subcores
    NW = NC * NS
    assert B % (8 * NW) == 0
    b_per_w = B // NW
    mesh = plsc.VectorSubcoreMesh(core_axis_name="c", subcore_axis_name="s")

    @functools.partial(
        pl.kernel,
        mesh=mesh,
        out_type=jax.ShapeDtypeStruct((B, _D), jnp.float32),
        scratch_types=[
            pltpu.VMEM((b_per_w,), jnp.int32),
            pltpu.VMEM((b_per_w, _D), jnp.float32),
            pltpu.SemaphoreType.DMA,
        ],
    )
    def gather_k(tbl_hbm, idx_hbm, out_hbm, idx_v, rows_v, sem):
        wid = lax.axis_index("s") * NC + lax.axis_index("c")
        base = wid * b_per_w
        pltpu.sync_copy(idx_hbm.at[pl.ds(base, b_per_w)], idx_v)
        pltpu.async_copy(tbl_hbm.at[idx_v], rows_v, sem).wait()
        pltpu.sync_copy(rows_v, out_hbm.at[pl.ds(base, b_per_w)])

    return gather_k


def _tc_body(Un, Tn, Kn, g_ref, resp_ref, W1_ref, b1_ref,
             W2_ref, b2_ref, W3_ref, b3_ref,
             logits_ref, last_ref,
             lm_ref, lmmu_ref, pre_ref, rden_ref, cm_ref, icnt_ref):
    B = Un * Tn
    C = 2048
    TC = C // Un
    f32 = jnp.float32
    Hn = W2_ref.shape[0]

    scale = 1.0 / Kn
    W1cat = jnp.concatenate(
        [jnp.broadcast_to(W1_ref[0:1, :] * scale, (Kn, Hn)),
         jnp.broadcast_to(W1_ref[1:2, :] * scale, (Kn, Hn)),
         jnp.broadcast_to(W1_ref[2:3, :] * scale, (Kn, Hn))], axis=0)
    W3cat = jnp.concatenate(
        [jnp.broadcast_to(W3_ref[:, 0:1], (Hn, Kn)),
         jnp.broadcast_to(W3_ref[:, 1:2], (Hn, Kn))], axis=1)
    b3cat = jnp.concatenate(
        [jnp.broadcast_to(b3_ref[0:1, 0:1], (1, Kn)),
         jnp.broadcast_to(b3_ref[0:1, 1:2], (1, Kn))], axis=1)

    ones_k = jnp.ones((Kn, Kn), f32)
    ones_uk = jnp.ones((Un, Kn), f32)
    rrow = lax.broadcasted_iota(jnp.int32, (C, TC), 0) // Un
    rcol = lax.broadcasted_iota(jnp.int32, (C, TC), 1)
    sel = (rrow == rcol).astype(f32)
    urow = lax.broadcasted_iota(jnp.int32, (C, Un), 0) % Un
    ucol = lax.broadcasted_iota(jnp.int32, (C, Un), 1)
    eye_u = (urow == ucol).astype(f32)

    b1 = b1_ref[...]
    b2 = b2_ref[...]

    for c in range(B // C):
        r0 = c * C
        rows = pl.ds(r0, C)
        mch = g_ref[rows, 0:Kn]
        rsel = jnp.dot(sel, resp_ref[pl.ds(c * TC, TC), :],
                       preferred_element_type=f32)
        r16 = jnp.dot(rsel * eye_u, ones_uk, preferred_element_type=f32)
        xfull = jnp.concatenate([g_ref[rows, Kn:3 * Kn], r16], axis=1)
        h = _gelu(jnp.dot(xfull, W1cat, preferred_element_type=f32) + b1)
        h = _gelu(jnp.dot(h, W2_ref[...], preferred_element_type=f32) + b2)
        o = jnp.dot(h, W3cat, preferred_element_type=f32) + b3cat
        mu16 = _gelu(o[:, 0:Kn])
        lv16 = jnp.minimum(_gelu(o[:, Kn:2 * Kn]), 1e8)
        lm16 = jnp.exp(-lv16)
        lm_ref[rows, :] = lm16
        lmmu_ref[rows, :] = lm16 * mu16
        cnt = jnp.dot(mch, ones_k, preferred_element_type=f32)
        icnt_ref[rows, :] = 1.0 / jnp.maximum(cnt, 1e-8)

    zeros = jnp.zeros((Un, Kn), f32)

    def bwd(i, carry):
        al, be = carry
        t = (Tn - 1) - i
        rows = pl.ds(pl.multiple_of(t * Un, Un), Un)
        m = g_ref[rows, 0:Kn] != 0.0
        lm = lm_ref[rows, :]
        lmmu = lmmu_ref[rows, :]
        num = lmmu + al * be
        den = lm + al
        rf = 1.0 / (1.0 + den)
        pre_ref[rows, :] = num
        rden_ref[rows, :] = rf
        al_new = den * rf
        be_new = num / den
        return (jnp.where(m, al_new, al), jnp.where(m, be_new, be))

    lax.fori_loop(0, Tn, bwd, (zeros, zeros), unroll=16)

    def fwd(t, curr):
        rows = pl.ds(pl.multiple_of(t * Un, Un), Un)
        mf = g_ref[rows, 0:Kn]
        cand = (curr + pre_ref[rows, :]) * rden_ref[rows, :]
        curr = jnp.where(mf != 0.0, cand, curr)
        cm_ref[rows, :] = curr * mf
        return curr

    last_ref[...] = lax.fori_loop(0, Tn, fwd, zeros, unroll=8)

    for c in range(B // C):
        rows = pl.ds(c * C, C)
        ssum = jnp.dot(cm_ref[rows, :], ones_k, preferred_element_type=f32)
        abil = ssum * icnt_ref[rows, :]
        d16 = g_ref[rows, Kn:2 * Kn]
        s16 = g_ref[rows, 2 * Kn:3 * Kn]
        logits_ref[rows, :] = (s16 * (abil - d16))[:, 0:1]


def kernel(mask, q_id, kmap, resp, diff_mu, disc_mu, W1, b1, W2, b2, W3, b3):
    Un, Tn = q_id.shape
    Qn, Kn = kmap.shape
    Hn = W1.shape[1]
    B = Un * Tn
    f32 = jnp.float32

    ci = jnp.arange(_D, dtype=jnp.int32)[None, :]
    kpad = jnp.pad(kmap, ((0, 0), (0, _D - Kn))).astype(f32)
    tbl = jnp.where(ci < Kn, kpad,
                    jnp.where(ci < 2 * Kn, diff_mu[:, None],
                              jnp.where(ci < 3 * Kn, disc_mu[:, None], 0.0)))
    idx = q_id.astype(jnp.int32).T.reshape(B)
    g = _make_sc_gather(Qn, B)(tbl, idx)

    resp_t = resp.astype(f32).T

    logits_col, last = pl.pallas_call(
        functools.partial(_tc_body, Un, Tn, Kn),
        out_shape=(jax.ShapeDtypeStruct((B, 1), f32),
                   jax.ShapeDtypeStruct((Un, Kn), f32)),
        scratch_shapes=[pltpu.VMEM((B, Kn), f32)] * 6,
    )(g, resp_t, W1, b1.reshape(1, Hn), W2, b2.reshape(1, Hn),
      W3, b3.reshape(1, 2))

    trial_logits = logits_col.reshape(Tn, Un).T
    return (trial_logits, last)


# speedup vs baseline: 1.0214x; 1.0005x over previous
import functools

import jax
import jax.numpy as jnp
from jax import lax
from jax.experimental import pallas as pl
from jax.experimental.pallas import tpu as pltpu
from jax.experimental.pallas import tpu_sc as plsc

_D = 128


def _gelu(x):
    return 0.5 * x * (1.0 + lax.erf(x * (2.0 ** -0.5)))


@functools.lru_cache(maxsize=None)
def _make_sc_gather(Qn, B):
    info = plsc.get_sparse_core_info()
    NC, NS = info.num_cores, info.num_subcores
    NW = NC * NS
    assert B % (8 * NW) == 0
    b_per_w = B // NW
    mesh = plsc.VectorSubcoreMesh(core_axis_name="c", subcore_axis_name="s")

    @functools.partial(
        pl.kernel,
        mesh=mesh,
        out_type=jax.ShapeDtypeStruct((B, _D), jnp.float32),
        scratch_types=[
            pltpu.VMEM((b_per_w,), jnp.int32),
            pltpu.VMEM((b_per_w, _D), jnp.float32),
            pltpu.SemaphoreType.DMA,
        ],
    )
    def gather_k(tbl_hbm, idx_hbm, out_hbm, idx_v, rows_v, sem):
        wid = lax.axis_index("s") * NC + lax.axis_index("c")
        base = wid * b_per_w
        pltpu.sync_copy(idx_hbm.at[pl.ds(base, b_per_w)], idx_v)
        pltpu.async_copy(tbl_hbm.at[idx_v], rows_v, sem).wait()
        pltpu.sync_copy(rows_v, out_hbm.at[pl.ds(base, b_per_w)])

    return gather_k


def _tc_body(Un, Tn, Kn, g_ref, resp_ref, W1_ref, b1_ref,
             W2_ref, b2_ref, W3_ref, b3_ref,
             logits_ref, last_ref,
             lm_ref, lmmu_ref, pre_ref, rden_ref, cm_ref, icnt_ref):
    B = Un * Tn
    C = 2048
    TC = C // Un
    f32 = jnp.float32
    Hn = W2_ref.shape[0]

    scale = 1.0 / Kn
    W1cat = jnp.concatenate(
        [jnp.broadcast_to(W1_ref[0:1, :] * scale, (Kn, Hn)),
         jnp.broadcast_to(W1_ref[1:2, :] * scale, (Kn, Hn)),
         jnp.broadcast_to(W1_ref[2:3, :] * scale, (Kn, Hn))], axis=0)
    W3cat = jnp.concatenate(
        [jnp.broadcast_to(W3_ref[:, 0:1], (Hn, Kn)),
         jnp.broadcast_to(W3_ref[:, 1:2], (Hn, Kn))], axis=1)
    b3cat = jnp.concatenate(
        [jnp.broadcast_to(b3_ref[0:1, 0:1], (1, Kn)),
         jnp.broadcast_to(b3_ref[0:1, 1:2], (1, Kn))], axis=1)

    ones_k = jnp.ones((Kn, Kn), f32)
    ones_uk = jnp.ones((Un, Kn), f32)
    rrow = lax.broadcasted_iota(jnp.int32, (C, TC), 0) // Un
    rcol = lax.broadcasted_iota(jnp.int32, (C, TC), 1)
    sel = (rrow == rcol).astype(f32)
    urow = lax.broadcasted_iota(jnp.int32, (C, Un), 0) % Un
    ucol = lax.broadcasted_iota(jnp.int32, (C, Un), 1)
    eye_u = (urow == ucol).astype(f32)

    b1 = b1_ref[...]
    b2 = b2_ref[...]

    for c in range(B // C):
        r0 = c * C
        rows = pl.ds(r0, C)
        mch = g_ref[rows, 0:Kn]
        rsel = jnp.dot(sel, resp_ref[pl.ds(c * TC, TC), :],
                       preferred_element_type=f32)
        r16 = jnp.dot(rsel * eye_u, ones_uk, preferred_element_type=f32)
        xfull = jnp.concatenate([g_ref[rows, Kn:3 * Kn], r16], axis=1)
        h = _gelu(jnp.dot(xfull, W1cat, preferred_element_type=f32) + b1)
        h = _gelu(jnp.dot(h, W2_ref[...], preferred_element_type=f32) + b2)
        o = jnp.dot(h, W3cat, preferred_element_type=f32) + b3cat
        mu16 = _gelu(o[:, 0:Kn])
        lv16 = jnp.minimum(_gelu(o[:, Kn:2 * Kn]), 1e8)
        lm16 = jnp.exp(-lv16)
        lm_ref[rows, :] = lm16
        lmmu_ref[rows, :] = lm16 * mu16
        cnt = jnp.dot(mch, ones_k, preferred_element_type=f32)
        icnt_ref[rows, :] = 1.0 / jnp.maximum(cnt, 1e-8)

    zeros = jnp.zeros((Un, Kn), f32)

    def bwd(i, carry):
        al, be = carry
        t = (Tn - 1) - i
        rows = pl.ds(pl.multiple_of(t * Un, Un), Un)
        m = g_ref[rows, 0:Kn] != 0.0
        lm = lm_ref[rows, :]
        lmmu = lmmu_ref[rows, :]
        num = lmmu + al * be
        den = lm + al
        rf = 1.0 / (1.0 + den)
        pre_ref[rows, :] = num
        rden_ref[rows, :] = rf
        al_new = den * rf
        be_new = num / den
        return (jnp.where(m, al_new, al), jnp.where(m, be_new, be))

    lax.fori_loop(0, Tn, bwd, (zeros, zeros), unroll=8)

    def fwd(t, curr):
        rows = pl.ds(pl.multiple_of(t * Un, Un), Un)
        mf = g_ref[rows, 0:Kn]
        cand = (curr + pre_ref[rows, :]) * rden_ref[rows, :]
        curr = jnp.where(mf != 0.0, cand, curr)
        cm_ref[rows, :] = curr * mf
        return curr

    last_ref[...] = lax.fori_loop(0, Tn, fwd, zeros, unroll=8)

    for c in range(B // C):
        rows = pl.ds(c * C, C)
        ssum = jnp.dot(cm_ref[rows, :], ones_k, preferred_element_type=f32)
        abil = ssum * icnt_ref[rows, :]
        d16 = g_ref[rows, Kn:2 * Kn]
        s16 = g_ref[rows, 2 * Kn:3 * Kn]
        logits_ref[rows, :] = (s16 * (abil - d16))[:, 0:1]


def kernel(mask, q_id, kmap, resp, diff_mu, disc_mu, W1, b1, W2, b2, W3, b3):
    Un, Tn = q_id.shape
    Qn, Kn = kmap.shape
    Hn = W1.shape[1]
    B = Un * Tn
    f32 = jnp.float32

    ci = jnp.arange(_D, dtype=jnp.int32)[None, :]
    kpad = jnp.pad(kmap, ((0, 0), (0, _D - Kn))).astype(f32)
    tbl = jnp.where(ci < Kn, kpad,
                    jnp.where(ci < 2 * Kn, diff_mu[:, None],
                              jnp.where(ci < 3 * Kn, disc_mu[:, None], 0.0)))
    idx = q_id.astype(jnp.int32).T.reshape(B)
    g = _make_sc_gather(Qn, B)(tbl, idx)

    resp_t = resp.astype(f32).T

    logits_col, last = pl.pallas_call(
        functools.partial(_tc_body, Un, Tn, Kn),
        out_shape=(jax.ShapeDtypeStruct((B, 1), f32),
                   jax.ShapeDtypeStruct((Un, Kn), f32)),
        scratch_shapes=[pltpu.VMEM((B, Kn), f32)] * 6,
    )(g, resp_t, W1, b1.reshape(1, Hn), W2, b2.reshape(1, Hn),
      W3, b3.reshape(1, 2))

    trial_logits = logits_col.reshape(Tn, Un).T
    return (trial_logits, last)
